# Initial kernel scaffold; baseline (speedup 1.0000x reference)
import jax
import jax.numpy as jnp
from jax.experimental import pallas as pl


def kernel(z, embedding_weight):
    raise NotImplementedError("write your pallas kernel here")


# speedup vs baseline: 1.5115x; 1.5115x over previous
import functools

import jax
import jax.numpy as jnp
from jax import lax
from jax.experimental import pallas as pl
from jax.experimental.pallas import tpu as pltpu
from jax.experimental.pallas import tpu_sc as plsc

N_CODES = 8192
DIM = 32
N_TOKENS = 65536
_BETA = 0.25

_TB = 256

_SC_CORES = 2
_SC_SUBCORES = 16
_NW = _SC_CORES * _SC_SUBCORES
_B_PER_W = N_TOKENS // _NW


def _dist_body(z_ref, et_ref, idx_ref, msum_ref):
    z = z_ref[...]
    et = et_ref[...]
    zsq = jnp.sum(z * z, axis=1, keepdims=True)
    esq = jnp.sum(et * et, axis=0, keepdims=True)
    m = lax.dot_general(z, et, (((1,), (0,)), ((), ())),
                        preferred_element_type=jnp.float32)
    d = (zsq + esq) - 2.0 * m
    half = N_CODES // 2
    d_lo, d_hi = d[:, :half], d[:, half:]
    ids = lax.broadcasted_iota(jnp.int32, d_lo.shape, 1)
    v_lo = jnp.min(d_lo, axis=1, keepdims=True)
    i_lo = jnp.min(jnp.where(d_lo == v_lo, ids, half), axis=1, keepdims=True)
    v_hi = jnp.min(d_hi, axis=1, keepdims=True)
    i_hi = jnp.min(jnp.where(d_hi == v_hi, ids, half), axis=1, keepdims=True) + half
    v_lo_bf = v_lo.astype(jnp.bfloat16).astype(jnp.float32)
    take_hi = v_hi < v_lo_bf
    idx_ref[...] = jnp.where(take_hi, i_hi, i_lo)
    d_pick = jnp.where(take_hi, v_hi, v_lo)

    partial = jnp.sum(d_pick, axis=(0, 1), keepdims=True)

    @pl.when(pl.program_id(0) == 0)
    def _():
        msum_ref[...] = partial

    @pl.when(pl.program_id(0) > 0)
    def _():
        msum_ref[...] += partial


_dist_call = pl.pallas_call(
    _dist_body,
    grid=(N_TOKENS // _TB,),
    in_specs=[
        pl.BlockSpec((_TB, DIM), lambda i: (i, 0)),
        pl.BlockSpec((DIM, N_CODES), lambda i: (0, 0)),
    ],
    out_specs=[
        pl.BlockSpec((_TB, 1), lambda i: (i, 0)),
        pl.BlockSpec((1, 1), lambda i: (0, 0)),
    ],
    out_shape=[
        jax.ShapeDtypeStruct((N_TOKENS, 1), jnp.int32),
        jax.ShapeDtypeStruct((1, 1), jnp.float32),
    ],
)


_PAD = 128
_CHUNK = 512
_N_CHUNKS = _B_PER_W // _CHUNK


@functools.cache
def _get_sc_gather():
    mesh = plsc.VectorSubcoreMesh(core_axis_name="c", subcore_axis_name="s")

    @functools.partial(
        pl.kernel,
        mesh=mesh,
        out_type=jax.ShapeDtypeStruct((N_TOKENS, _PAD), jnp.float32),
        scratch_types=[
            pltpu.VMEM((_B_PER_W,), jnp.int32),
            pltpu.VMEM((_CHUNK, _PAD), jnp.float32),
            pltpu.SemaphoreType.DMA,
        ],
    )
    def _sc_gather(table_hbm, idx_hbm, out_hbm, idx_v, rows_v, sem):
        wid = lax.axis_index("s") * _SC_CORES + lax.axis_index("c")
        base = wid * _B_PER_W
        pltpu.sync_copy(idx_hbm.at[pl.ds(base, _B_PER_W)], idx_v)
        for c in range(_N_CHUNKS):
            pltpu.async_copy(
                table_hbm.at[idx_v.at[pl.ds(c * _CHUNK, _CHUNK)]], rows_v, sem
            ).wait()
            pltpu.sync_copy(rows_v, out_hbm.at[pl.ds(base + c * _CHUNK, _CHUNK)])

    return _sc_gather


def kernel(z, embedding_weight):
    et = embedding_weight.T
    idx2, msum = _dist_call(z, et)
    table = jnp.pad(embedding_weight, ((0, 0), (0, _PAD - DIM)))
    z_q_pad = _get_sc_gather()(table, idx2.reshape(N_TOKENS))
    z_q = z_q_pad[:, :DIM]
    loss = msum[0, 0] * ((1.0 + _BETA) / (N_TOKENS * DIM))
    return (z_q, loss, idx2)
